# baseline (speedup 1.0000x reference)
import functools

import jax
import jax.numpy as jnp
from jax.experimental import pallas as pl
from jax.experimental.pallas import tpu as pltpu


def _mlp_body(n_nodes, x_ref, valid_ref, w1_ref, b1_ref, w2_ref, b2_ref, out_ref):
    hs = []
    for n in range(n_nodes):
        xn = jnp.maximum(x_ref[32 * n:32 * (n + 1), :], 0.0).astype(jnp.bfloat16)
        hn = jnp.dot(w1_ref[n], xn, preferred_element_type=jnp.float32)
        hn = jnp.maximum(hn + b1_ref[n][:, None], 0.0).astype(jnp.bfloat16)
        hs.append(hn)
    hid = jnp.concatenate(hs, axis=0)
    out = jnp.dot(w2_ref[...], hid, preferred_element_type=jnp.float32)
    out = out + b2_ref[...]
    out = jnp.where(valid_ref[...].astype(jnp.int32) > 0, out, 0.0)
    out_ref[...] = out.reshape(out_ref.shape)


@functools.partial(jax.jit, static_argnames=("block_lanes",))
def _run(h, valid, W1, b1, W2, b2, block_lanes=4096):
    B, N, C = h.shape
    Wh = W1.shape[2]

    xT = h.transpose(1, 2, 0).reshape(N * C, B)
    validT = valid.transpose(1, 0).astype(jnp.int8)

    W1T = W1.transpose(0, 2, 1).astype(jnp.bfloat16)

    eye_n = jnp.eye(N, dtype=W2.dtype)
    W2row = (eye_n[:, :, None] * W2[None, :, :, 0]).reshape(N, N * Wh).astype(jnp.bfloat16)

    grid = (B // block_lanes,)
    outT = pl.pallas_call(
        functools.partial(_mlp_body, N),
        grid=grid,
        in_specs=[
            pl.BlockSpec((N * C, block_lanes), lambda j: (0, j)),
            pl.BlockSpec((N, block_lanes), lambda j: (0, j)),
            pl.BlockSpec((N, Wh, C), lambda j: (0, 0, 0)),
            pl.BlockSpec((N, Wh), lambda j: (0, 0)),
            pl.BlockSpec((N, N * Wh), lambda j: (0, 0)),
            pl.BlockSpec((N, 1), lambda j: (0, 0)),
        ],
        out_specs=pl.BlockSpec((N, block_lanes // 128, 128), lambda j: (0, j, 0)),
        out_shape=jax.ShapeDtypeStruct((N, B // 128, 128), jnp.float32),
        compiler_params=pltpu.CompilerParams(
            dimension_semantics=("arbitrary",),
        ),
    )(xT, validT, W1T, b1, W2row, b2)
    return outT.transpose(1, 2, 0).reshape(B, N)[:, :, None]


def kernel(h, valid, W1, b1, W2, b2):
    return _run(h, valid, W1, b1, W2, b2)


# speedup vs baseline: 1.0772x; 1.0772x over previous
import functools

import jax
import jax.numpy as jnp
from jax.experimental import pallas as pl
from jax.experimental.pallas import tpu as pltpu


def _mlp_body(n_nodes, x_ref, valid_ref, w1_ref, b1_ref, w2_ref, b2_ref, out_ref):
    out = None
    for g in range(n_nodes // 4):
        hs = []
        for j in range(4):
            n = 4 * g + j
            xn = jnp.maximum(x_ref[32 * n:32 * (n + 1), :], 0.0).astype(jnp.bfloat16)
            hn = jnp.dot(w1_ref[n], xn, preferred_element_type=jnp.float32)
            hn = jnp.maximum(hn + b1_ref[n][:, None], 0.0).astype(jnp.bfloat16)
            hs.append(hn)
        hid_g = jnp.concatenate(hs, axis=0)
        cg = jnp.dot(w2_ref[:, 128 * g:128 * (g + 1)], hid_g,
                     preferred_element_type=jnp.float32)
        out = cg if out is None else out + cg
    out = out + b2_ref[...]
    out = jnp.where(valid_ref[...] > 0, out, 0.0)
    out_ref[...] = out.reshape(out_ref.shape)


@functools.partial(jax.jit, static_argnames=("block_lanes",))
def _run(h, valid, W1, b1, W2, b2, block_lanes=8192):
    B, N, C = h.shape
    Wh = W1.shape[2]

    xT = h.transpose(1, 2, 0).reshape(N * C, B)
    validT = valid.transpose(1, 0)

    W1T = W1.transpose(0, 2, 1).astype(jnp.bfloat16)

    eye_n = jnp.eye(N, dtype=W2.dtype)
    W2row = (eye_n[:, :, None] * W2[None, :, :, 0]).reshape(N, N * Wh).astype(jnp.bfloat16)

    grid = (B // block_lanes,)
    outT = pl.pallas_call(
        functools.partial(_mlp_body, N),
        grid=grid,
        in_specs=[
            pl.BlockSpec((N * C, block_lanes), lambda j: (0, j)),
            pl.BlockSpec((N, block_lanes), lambda j: (0, j)),
            pl.BlockSpec((N, Wh, C), lambda j: (0, 0, 0)),
            pl.BlockSpec((N, Wh), lambda j: (0, 0)),
            pl.BlockSpec((N, N * Wh), lambda j: (0, 0)),
            pl.BlockSpec((N, 1), lambda j: (0, 0)),
        ],
        out_specs=pl.BlockSpec((N, block_lanes // 128, 128), lambda j: (0, j, 0)),
        out_shape=jax.ShapeDtypeStruct((N, B // 128, 128), jnp.float32),
        compiler_params=pltpu.CompilerParams(
            dimension_semantics=("arbitrary",),
        ),
    )(xT, validT, W1T, b1, W2row, b2)
    return outT.transpose(1, 2, 0).reshape(B, N)[:, :, None]


def kernel(h, valid, W1, b1, W2, b2):
    return _run(h, valid, W1, b1, W2, b2)


# speedup vs baseline: 1.0775x; 1.0003x over previous
import functools

import jax
import jax.numpy as jnp
from jax.experimental import pallas as pl
from jax.experimental.pallas import tpu as pltpu


def _mlp_body(n_nodes, x_ref, valid_ref, w1_ref, b1_ref, w2_ref, b2_ref, out_ref):
    out = None
    for g in range(n_nodes // 4):
        hs = []
        for j in range(4):
            n = 4 * g + j
            xn = jnp.maximum(x_ref[32 * n:32 * (n + 1), :], 0.0).astype(jnp.bfloat16)
            hn = jnp.dot(w1_ref[n], xn, preferred_element_type=jnp.float32)
            hn = jnp.maximum(hn + b1_ref[n][:, None], 0.0).astype(jnp.bfloat16)
            hs.append(hn)
        hid_g = jnp.concatenate(hs, axis=0)
        cg = jnp.dot(w2_ref[:, 128 * g:128 * (g + 1)], hid_g,
                     preferred_element_type=jnp.float32)
        out = cg if out is None else out + cg
    out = out + b2_ref[...]
    out = jnp.where(valid_ref[...] > 0, out, 0.0)
    out_ref[...] = out.reshape(out_ref.shape)


@functools.partial(jax.jit, static_argnames=("block_lanes",))
def _run(h, valid, W1, b1, W2, b2, block_lanes=8192):
    B, N, C = h.shape
    Wh = W1.shape[2]

    xT = h.transpose(1, 2, 0).reshape(N * C, B)
    validT = valid.transpose(1, 0)

    W1T = W1.transpose(0, 2, 1).astype(jnp.bfloat16)

    eye_n = jnp.eye(N, dtype=W2.dtype)
    W2row = (eye_n[:, :, None] * W2[None, :, :, 0]).reshape(N, N * Wh).astype(jnp.bfloat16)

    grid = (B // block_lanes,)
    outT = pl.pallas_call(
        functools.partial(_mlp_body, N),
        grid=grid,
        in_specs=[
            pl.BlockSpec((N * C, block_lanes), lambda j: (0, j)),
            pl.BlockSpec((N, block_lanes), lambda j: (0, j)),
            pl.BlockSpec((N, Wh, C), lambda j: (0, 0, 0)),
            pl.BlockSpec((N, Wh), lambda j: (0, 0)),
            pl.BlockSpec((N, N * Wh), lambda j: (0, 0)),
            pl.BlockSpec((N, 1), lambda j: (0, 0)),
        ],
        out_specs=pl.BlockSpec((N, block_lanes // 128, 128), lambda j: (0, j, 0)),
        out_shape=jax.ShapeDtypeStruct((N, B // 128, 128), jnp.float32),
        compiler_params=pltpu.CompilerParams(
            dimension_semantics=("parallel",),
        ),
    )(xT, validT, W1T, b1, W2row, b2)
    return outT.transpose(1, 2, 0).reshape(B, N)[:, :, None]


def kernel(h, valid, W1, b1, W2, b2):
    return _run(h, valid, W1, b1, W2, b2)
